# baseline (speedup 1.0000x reference)
import functools

import jax
import jax.numpy as jnp
from jax import lax
from jax.experimental import pallas as pl
from jax.experimental.pallas import tpu as pltpu
from jax.experimental.pallas import tpu_sc as plsc

EPS = 1e-06
LN2 = 0.6931471805599453
NB = 32
ROW = 512 * 512 * 2
ETROW = ROW // 2
CH = 16384
NCH = ROW // CH
STEPS = CH // 32


def _fastlog(x):
    bits = plsc.bitcast(x, jnp.int32)
    ef = ((bits >> 23) - 127).astype(jnp.float32)
    m = plsc.bitcast((bits & 0x007FFFFF) | 0x3F800000, jnp.float32)
    tt = (m - 1.0) / (m + 1.0)
    t2 = tt * tt
    p = 2.0 / 5.0
    p = 2.0 / 3.0 + p * t2
    p = 2.0 + p * t2
    return ef * LN2 + tt * p


def _splat(s):
    return jnp.full((16,), s, dtype=jnp.float32)


QK1 = LN2 * 2.0 ** -23
QK2 = LN2 * 127.0
QK3 = LN2 * 0.3466
QU = 2.0 ** -23


def _qlog_terms(x):
    bits = plsc.bitcast(x, jnp.int32)
    bf = bits.astype(jnp.float32)
    u = (bits & 0x007FFFFF).astype(jnp.float32) * QU
    return bf, u * (1.0 - u)


def _sc_body(e0_hbm, et_hbm, t_hbm, tab_hbm, out_hbm, t_v, qrow,
             e0_a, et_a, out_a, e0_b, et_b, out_b,
             sin_a, sin_b, sout_a, sout_b):
    wid = lax.axis_index("s") * 2 + lax.axis_index("c")
    base = wid * ROW
    etbase = wid * ETROW
    bufs = ((e0_a, et_a, out_a, sin_a, sout_a), (e0_b, et_b, out_b, sin_b, sout_b))

    pltpu.sync_copy(t_hbm, t_v)
    tw = plsc.load_gather(t_v, [jnp.full((16,), wid, dtype=jnp.int32)])[0]

    def in_copy(ci, bi):
        e0b, etb, _, sin, _ = bufs[bi]
        off = base + ci * CH
        return (pltpu.make_async_copy(e0_hbm.at[pl.ds(off, CH)], e0b, sin),
                pltpu.make_async_copy(
                    et_hbm.at[pl.ds(etbase + ci * (CH // 2), CH // 2)], etb, sin))

    def out_copy(ci, bi):
        _, _, outb, _, sout = bufs[bi]
        return pltpu.make_async_copy(outb, out_hbm.at[pl.ds(base + ci * CH, CH)], sout)

    @pl.when(tw == 0)
    def _copy_row():
        def copy_chunk(ci, carry):
            off = base + ci * CH
            pltpu.sync_copy(e0_hbm.at[pl.ds(off, CH)], e0_a)
            pltpu.sync_copy(e0_a, out_hbm.at[pl.ds(off, CH)])
            return carry
        lax.fori_loop(0, NCH, copy_chunk, 0)

    @pl.when(tw != 0)
    def _compute_row():
        pltpu.sync_copy(tab_hbm.at[pl.ds(tw * 16, 16)], qrow)
        qv = qrow[pl.ds(0, 16)]
        a0v = _splat(qv[4] - qv[6])
        b0v = _splat(qv[6] + EPS)
        l00 = _fastlog(_splat(qv[0] + EPS)) - QK2
        l01 = _fastlog(_splat(qv[1] + EPS)) - QK2
        l10 = _fastlog(_splat(qv[2] + EPS)) - QK2
        l11 = _fastlog(_splat(qv[3] + EPS)) - QK2

        def compute_chunk(bi):
            e0b, etb, outb, _, _ = bufs[bi]

            @plsc.parallel_loop(0, STEPS // 8, step=1, unroll=1)
            def step(i):
                row = i >> 2
                jb = i & 3
                off0 = row * 1024 + jb * 256
                offe = (row >> 3) * 4096 + jb * 1024 + (row & 7) * 128
                for g in range(8):
                    x0 = e0b[pl.ds(off0 + g * 16, 16)]
                    x1 = e0b[pl.ds(off0 + g * 16 + 128, 16)]
                    etx = etb[pl.ds(offe + g * 16, 16)]
                    s0 = 1.0 / (1.0 + jnp.exp(x1 - x0))
                    f20 = s0 * a0v + b0v
                    bf0, q0 = _qlog_terms(f20)
                    bf1, q1 = _qlog_terms((1.0 + 2.0 * EPS) - f20)
                    m = etx == 0
                    outb[pl.ds(off0 + g * 16, 16)] = (
                        bf0 * QK1 + jnp.where(m, l00, l10) + q0 * QK3)
                    outb[pl.ds(off0 + g * 16 + 128, 16)] = (
                        bf1 * QK1 + jnp.where(m, l01, l11) + q1 * QK3)

        for d in in_copy(0, 0):
            d.start()
        for d in in_copy(1, 1):
            d.start()

        def pipe(outer, carry):
            for bi in range(2):
                ci = outer * 2 + bi
                for d in in_copy(ci, bi):
                    d.wait()

                @pl.when(ci >= 2)
                def _drain():
                    out_copy(ci - 2, bi).wait()

                compute_chunk(bi)
                out_copy(ci, bi).start()

                @pl.when(ci + 2 < NCH)
                def _next():
                    for d in in_copy(ci + 2, bi):
                        d.start()
            return carry

        lax.fori_loop(0, NCH // 2, pipe, 0)
        out_copy(NCH - 2, 0).wait()
        out_copy(NCH - 1, 1).wait()


@functools.partial(jax.jit, static_argnames=())
def _run(e0f, etf, tt, tab):
    mesh = plsc.VectorSubcoreMesh(core_axis_name="c", subcore_axis_name="s",
                                  num_cores=2, num_subcores=16)
    return pl.kernel(
        _sc_body,
        out_type=jax.ShapeDtypeStruct((NB * ROW,), jnp.float32),
        mesh=mesh,
        scratch_types=[
            pltpu.VMEM((NB,), jnp.int32),
            pltpu.VMEM((16,), jnp.float32),
            pltpu.VMEM((CH,), jnp.float32),
            pltpu.VMEM((CH // 2,), jnp.int32),
            pltpu.VMEM((CH,), jnp.float32),
            pltpu.VMEM((CH,), jnp.float32),
            pltpu.VMEM((CH // 2,), jnp.int32),
            pltpu.VMEM((CH,), jnp.float32),
            pltpu.SemaphoreType.DMA,
            pltpu.SemaphoreType.DMA,
            pltpu.SemaphoreType.DMA,
            pltpu.SemaphoreType.DMA,
        ],
        compiler_params=pltpu.CompilerParams(needs_layout_passes=False),
    )(e0f, etf, tt, tab)


def kernel(e_0, e_t, t, q_one_step_transposed, q_mats):
    b, n = e_0.shape[0], e_0.shape[1]
    tidx = jnp.arange(q_one_step_transposed.shape[0], dtype=jnp.int32)
    tab = jnp.concatenate(
        [q_one_step_transposed.reshape(-1, 4), q_mats[tidx - 1].reshape(-1, 4),
         jnp.zeros((q_one_step_transposed.shape[0], 8), jnp.float32)],
        axis=1,
    ).reshape(-1)
    e0f = e_0.reshape(b, n, n // 128, 128, 2).transpose(0, 1, 2, 4, 3).reshape(-1)
    etf = e_t.reshape(b, n // 8, 8, n // 128, 128).transpose(0, 1, 3, 2, 4).reshape(-1)
    out = _run(e0f, etf, t.reshape(b).astype(jnp.int32), tab)
    return (out.reshape(b, n, n // 128, 2, 128)
               .transpose(0, 1, 2, 4, 3)
               .reshape(b, n, n, 2))


# speedup vs baseline: 1.0276x; 1.0276x over previous
import functools

import jax
import jax.numpy as jnp
from jax import lax
from jax.experimental import pallas as pl
from jax.experimental.pallas import tpu as pltpu
from jax.experimental.pallas import tpu_sc as plsc

EPS = 1e-06
LN2 = 0.6931471805599453
NB = 32
ROW = 512 * 512 * 2
ETROW = ROW // 2
CH = 16384
NCH = ROW // CH
STEPS = CH // 32


def _fastlog(x):
    bits = plsc.bitcast(x, jnp.int32)
    ef = ((bits >> 23) - 127).astype(jnp.float32)
    m = plsc.bitcast((bits & 0x007FFFFF) | 0x3F800000, jnp.float32)
    tt = (m - 1.0) / (m + 1.0)
    t2 = tt * tt
    p = 2.0 / 5.0
    p = 2.0 / 3.0 + p * t2
    p = 2.0 + p * t2
    return ef * LN2 + tt * p


def _splat(s):
    return jnp.full((16,), s, dtype=jnp.float32)


QK1 = LN2 * 2.0 ** -23
QK2 = LN2 * 127.0
QK3 = LN2 * 0.3466
QU = 2.0 ** -23


def _qlog_terms(x):
    bits = plsc.bitcast(x, jnp.int32)
    bf = bits.astype(jnp.float32)
    u = (bits & 0x007FFFFF).astype(jnp.float32) * QU
    return bf, u * (1.0 - u)


def _sc_body(e0_hbm, et_hbm, t_hbm, tab_hbm, out_hbm, t_v, qrow,
             e0_a, et_a, out_a, e0_b, et_b, out_b,
             sin_a, sin_b, sout_a, sout_b):
    wid = lax.axis_index("s") * 2 + lax.axis_index("c")
    base = wid * ROW
    etbase = wid * ETROW
    bufs = ((e0_a, et_a, out_a, sin_a, sout_a), (e0_b, et_b, out_b, sin_b, sout_b))

    pltpu.sync_copy(t_hbm, t_v)
    tw = plsc.load_gather(t_v, [jnp.full((16,), wid, dtype=jnp.int32)])[0]

    def in_copy(ci, bi):
        e0b, etb, _, sin, _ = bufs[bi]
        off = base + ci * CH
        return (pltpu.make_async_copy(e0_hbm.at[pl.ds(off, CH)], e0b, sin),
                pltpu.make_async_copy(
                    et_hbm.at[pl.ds(etbase + ci * (CH // 2), CH // 2)], etb, sin))

    def out_copy(ci, bi):
        _, _, outb, _, sout = bufs[bi]
        return pltpu.make_async_copy(outb, out_hbm.at[pl.ds(base + ci * CH, CH)], sout)

    @pl.when(tw == 0)
    def _copy_row():
        def copy_chunk(ci, carry):
            off = base + ci * CH
            pltpu.sync_copy(e0_hbm.at[pl.ds(off, CH)], e0_a)
            pltpu.sync_copy(e0_a, out_hbm.at[pl.ds(off, CH)])
            return carry
        lax.fori_loop(0, NCH, copy_chunk, 0)

    @pl.when(tw != 0)
    def _compute_row():
        pltpu.sync_copy(tab_hbm.at[pl.ds(tw * 16, 16)], qrow)
        qv = qrow[pl.ds(0, 16)]
        a0v = _splat(qv[4] - qv[6])
        b0v = _splat(qv[6] + EPS)
        l00 = _fastlog(_splat(qv[0] + EPS)) - QK2
        l01 = _fastlog(_splat(qv[1] + EPS)) - QK2
        l10 = _fastlog(_splat(qv[2] + EPS)) - QK2
        l11 = _fastlog(_splat(qv[3] + EPS)) - QK2

        def compute_chunk(bi):
            e0b, etb, outb, _, _ = bufs[bi]

            @plsc.parallel_loop(0, STEPS // 8, step=1, unroll=2)
            def step(i):
                row = i >> 2
                jb = i & 3
                off0 = row * 1024 + jb * 256
                offe = (row >> 3) * 4096 + jb * 1024 + (row & 7) * 128
                for g in range(8):
                    x0 = e0b[pl.ds(off0 + g * 16, 16)]
                    x1 = e0b[pl.ds(off0 + g * 16 + 128, 16)]
                    etx = etb[pl.ds(offe + g * 16, 16)]
                    s0 = 1.0 / (1.0 + jnp.exp(x1 - x0))
                    f20 = s0 * a0v + b0v
                    bf0, q0 = _qlog_terms(f20)
                    bf1, q1 = _qlog_terms((1.0 + 2.0 * EPS) - f20)
                    m = etx == 0
                    outb[pl.ds(off0 + g * 16, 16)] = (
                        bf0 * QK1 + jnp.where(m, l00, l10) + q0 * QK3)
                    outb[pl.ds(off0 + g * 16 + 128, 16)] = (
                        bf1 * QK1 + jnp.where(m, l01, l11) + q1 * QK3)

        for d in in_copy(0, 0):
            d.start()
        for d in in_copy(1, 1):
            d.start()

        def pipe(outer, carry):
            for bi in range(2):
                ci = outer * 2 + bi
                for d in in_copy(ci, bi):
                    d.wait()

                @pl.when(ci >= 2)
                def _drain():
                    out_copy(ci - 2, bi).wait()

                compute_chunk(bi)
                out_copy(ci, bi).start()

                @pl.when(ci + 2 < NCH)
                def _next():
                    for d in in_copy(ci + 2, bi):
                        d.start()
            return carry

        lax.fori_loop(0, NCH // 2, pipe, 0)
        out_copy(NCH - 2, 0).wait()
        out_copy(NCH - 1, 1).wait()


@functools.partial(jax.jit, static_argnames=())
def _run(e0f, etf, tt, tab):
    mesh = plsc.VectorSubcoreMesh(core_axis_name="c", subcore_axis_name="s",
                                  num_cores=2, num_subcores=16)
    return pl.kernel(
        _sc_body,
        out_type=jax.ShapeDtypeStruct((NB * ROW,), jnp.float32),
        mesh=mesh,
        scratch_types=[
            pltpu.VMEM((NB,), jnp.int32),
            pltpu.VMEM((16,), jnp.float32),
            pltpu.VMEM((CH,), jnp.float32),
            pltpu.VMEM((CH // 2,), jnp.int32),
            pltpu.VMEM((CH,), jnp.float32),
            pltpu.VMEM((CH,), jnp.float32),
            pltpu.VMEM((CH // 2,), jnp.int32),
            pltpu.VMEM((CH,), jnp.float32),
            pltpu.SemaphoreType.DMA,
            pltpu.SemaphoreType.DMA,
            pltpu.SemaphoreType.DMA,
            pltpu.SemaphoreType.DMA,
        ],
        compiler_params=pltpu.CompilerParams(needs_layout_passes=False),
    )(e0f, etf, tt, tab)


def kernel(e_0, e_t, t, q_one_step_transposed, q_mats):
    b, n = e_0.shape[0], e_0.shape[1]
    tidx = jnp.arange(q_one_step_transposed.shape[0], dtype=jnp.int32)
    tab = jnp.concatenate(
        [q_one_step_transposed.reshape(-1, 4), q_mats[tidx - 1].reshape(-1, 4),
         jnp.zeros((q_one_step_transposed.shape[0], 8), jnp.float32)],
        axis=1,
    ).reshape(-1)
    e0f = e_0.reshape(b, n, n // 128, 128, 2).transpose(0, 1, 2, 4, 3).reshape(-1)
    etf = e_t.reshape(b, n // 8, 8, n // 128, 128).transpose(0, 1, 3, 2, 4).reshape(-1)
    out = _run(e0f, etf, t.reshape(b).astype(jnp.int32), tab)
    return (out.reshape(b, n, n // 128, 2, 128)
               .transpose(0, 1, 2, 4, 3)
               .reshape(b, n, n, 2))
